# Initial kernel scaffold; baseline (speedup 1.0000x reference)
import jax
import jax.numpy as jnp
from jax.experimental import pallas as pl


def kernel(i, j, fluidPosition, fluidVelocity, fluidDensity, fluidArea, fluidPressure, fluidDistances, fluidRadialDistances):
    raise NotImplementedError("write your pallas kernel here")


# speedup vs baseline: 58.7684x; 58.7684x over previous
import functools

import numpy as np
import jax
import jax.numpy as jnp
from jax import lax
from jax.experimental import pallas as pl
from jax.experimental.pallas import tpu as pltpu, tpu_sc as plsc

SUPPORT = 0.05
REST_DENSITY = 1000.0
ALPHA = 0.01
DELTA = 0.1
C0 = 10.0 * np.sqrt(2.0 * 9.81 * 0.3)
EPS = SUPPORT ** 2 * 0.1
KGC = 7.0 / (np.pi * SUPPORT * SUPPORT)
DDC = SUPPORT * DELTA * C0
VC = SUPPORT * ALPHA * C0

NC, NS, L = 2, 16, 16
NW = NC * NS
CH = 128
CE = 112
G = CH // L

f32 = jnp.float32
i32 = jnp.int32

_CP = pltpu.CompilerParams(needs_layout_passes=False, use_tc_tiling_on_sc=False)


def _lane(g):
    return lax.iota(i32, L) + g * L


def _col(c):
    return jnp.full((L,), c, i32)


def _lg(ref, ln, c):
    return plsc.load_gather(ref, [ln, _col(c)])


def _gradw(q, dx, dy):
    t = jnp.maximum(1.0 - q, 0.0)
    t3 = (t * t) * t
    dWdq = (KGC * (-20.0 * q)) * t3
    w = dWdq / SUPPORT
    return w * dx, w * dy


def _extract(vec16, lane_scalar):
    sel = jnp.where(lax.iota(i32, L) == lane_scalar, vec16, jnp.full((L,), -1, i32))
    return lax.reduce_max(sel, axes=(0,))


def _make_e1(NPG, mesh):
    @functools.partial(
        pl.kernel,
        out_type=jax.ShapeDtypeStruct((NC, NPG, 8), f32),
        mesh=mesh,
        compiler_params=_CP,
        scratch_types=[
            pltpu.VMEM((CE,), i32), pltpu.VMEM((CE,), i32),
            pltpu.VMEM((CE,), f32), pltpu.VMEM((CE,), f32), pltpu.VMEM((CE,), f32),
            pltpu.VMEM((CE, 8), f32), pltpu.VMEM((CE, 8), f32),
            pltpu.VMEM((CE, 16), f32),
            pltpu.VMEM((CH,), i32),
            pltpu.VMEM((CH, 8), f32),
            pltpu.VMEM((48,), i32),
            pltpu.VMEM_SHARED((NPG, 8), f32),
        ],
    )
    def e1(ip, jp, qp, dxp, dyp, t1, cuts, zin, out,
           ib, jb, qb, dxb, dyb, gi, gj, cb, nidb, segb, cutv, acc):
        cid = lax.axis_index("c")
        sid = lax.axis_index("s")
        wid = sid * NC + cid
        NZ = NPG // NS
        GROW = NPG - 16
        pltpu.sync_copy(zin, acc.at[pl.ds(sid * NZ, NZ)])
        pltpu.sync_copy(cuts, cutv)
        zv = jnp.zeros((L,), f32)
        for g in range(CE // L):
            ln = _lane(g)
            for c in range(4, 16):
                plsc.store_scatter(cb, [ln, _col(c)], zv)
        plsc.subcore_barrier()

        v0 = cutv[pl.ds(0, L)]
        v1 = cutv[pl.ds(16, L)]
        v2 = cutv[pl.ds(32, L)]

        def cut_at(widx):
            blk = widx // L
            lane = widx - blk * L
            c0 = _extract(v0, jnp.where(blk == 0, lane, -2))
            c1 = _extract(v1, jnp.where(blk == 1, lane, -2))
            c2 = _extract(v2, jnp.where(blk == 2, lane, -2))
            return lax.max(c0, lax.max(c1, c2))

        s0 = cut_at(wid)
        s1 = cut_at(wid + 1)
        b0 = (s0 // CE) * CE
        trips = (s1 - b0 + (CE - 1)) // CE
        trips = jnp.where(s1 > s0, trips, 0)

        lane0 = lax.iota(i32, L)
        grow_v = jnp.full((L,), GROW, i32)
        stmask = lane0 < 8
        zvec = jnp.zeros((L,), i32)
        onev = jnp.full((L,), 1, i32)

        def chunk(t, carry):
            A, prev = carry
            base = b0 + t * CE
            pltpu.sync_copy(ip.at[pl.ds(base, CE)], ib)
            pltpu.sync_copy(jp.at[pl.ds(base, CE)], jb)
            pltpu.sync_copy(qp.at[pl.ds(base, CE)], qb)
            pltpu.sync_copy(dxp.at[pl.ds(base, CE)], dxb)
            pltpu.sync_copy(dyp.at[pl.ds(base, CE)], dyb)
            pltpu.sync_copy(t1.at[ib], gi)
            pltpu.sync_copy(t1.at[jb], gj)
            for g in range(G):
                plsc.store_scatter(nidb, [_lane(g)], grow_v)
            s0v = zvec + s0
            s1v = zvec + s1
            bv = zvec + base
            for g in range(CE // L):
                s = g * L
                ln = _lane(g)
                q = qb[pl.ds(s, L)]
                gwx, gwy = _gradw(q, dxb[pl.ds(s, L)], dyb[pl.ds(s, L)])
                pxi = _lg(gi, ln, 0); pyi = _lg(gi, ln, 1)
                pxj = _lg(gj, ln, 0); pyj = _lg(gj, ln, 1); vj = _lg(gj, ln, 3)
                rbx = pxj - pxi
                rby = pyj - pyi
                plsc.store_scatter(cb, [ln, _col(0)], (rbx * gwx) * vj)
                plsc.store_scatter(cb, [ln, _col(1)], (rbx * gwy) * vj)
                plsc.store_scatter(cb, [ln, _col(2)], (rby * gwx) * vj)
                plsc.store_scatter(cb, [ln, _col(3)], (rby * gwy) * vj)
            r = jnp.zeros((L,), i32)
            for e in range(CE):
                ecol = _col(e)
                ie = plsc.load_gather(ib, [ecol])
                eidx = bv + e
                valid = jnp.logical_and(eidx >= s0v, eidx < s1v)
                ve = plsc.load_gather(cb, [ecol, lane0])
                ve = jnp.where(valid, ve, zv)
                bnd = ie != prev
                A = jnp.where(bnd, ve, A + ve)
                r = r + jnp.where(bnd, onev, zvec)
                plsc.store_scatter(nidb, [r], ie, mask=valid)
                plsc.store_scatter(segb, [r, lane0], A,
                                   mask=jnp.logical_and(valid, stmask))
                prev = ie
            pltpu.sync_copy(segb, acc.at[nidb])
            return A, prev

        lax.fori_loop(0, trips, chunk,
                      (jnp.zeros((L,), f32), jnp.full((L,), -1, i32)))
        plsc.subcore_barrier()
        pltpu.sync_copy(acc.at[pl.ds(sid * NZ, NZ)], out.at[cid, pl.ds(sid * NZ, NZ)])

    return e1


def _make_n1(NPG, KN, mesh):
    @functools.partial(
        pl.kernel,
        out_type=jax.ShapeDtypeStruct((NPG, 8), f32),
        mesh=mesh,
        compiler_params=_CP,
        scratch_types=[
            pltpu.VMEM((CH, 8), f32), pltpu.VMEM((CH, 8), f32),
            pltpu.VMEM((CH, 8), f32),
        ],
    )
    def n1(p01, tbpre, tb_out, pa, pb, tb):
        cid = lax.axis_index("c")
        sid = lax.axis_index("s")
        wid = sid * NC + cid

        def chunk(k, carry):
            r0 = (wid * KN + k) * CH
            pltpu.sync_copy(p01.at[0, pl.ds(r0, CH)], pa)
            pltpu.sync_copy(p01.at[1, pl.ds(r0, CH)], pb)
            pltpu.sync_copy(tbpre.at[pl.ds(r0, CH)], tb)
            for g in range(G):
                ln = _lane(g)
                a = _lg(pa, ln, 0) + _lg(pb, ln, 0)
                b = _lg(pa, ln, 1) + _lg(pb, ln, 1)
                c = _lg(pa, ln, 2) + _lg(pb, ln, 2)
                d = _lg(pa, ln, 3) + _lg(pb, ln, 3)
                det = a * d - b * c
                safe = jnp.abs(det) > 1e-7
                inv = jnp.where(safe, 1.0 / jnp.where(safe, det, 1.0), 0.0)
                plsc.store_scatter(tb, [ln, _col(0)], d * inv)
                plsc.store_scatter(tb, [ln, _col(1)], (-b) * inv)
                plsc.store_scatter(tb, [ln, _col(2)], (-c) * inv)
                plsc.store_scatter(tb, [ln, _col(3)], a * inv)
            pltpu.sync_copy(tb, tb_out.at[pl.ds(r0, CH)])
            return carry

        lax.fori_loop(0, KN, chunk, 0)

    return n1


def _make_eb(NP, T, mesh):
    @functools.partial(
        pl.kernel,
        out_type=jax.ShapeDtypeStruct((NC, NP, 8), f32),
        mesh=mesh,
        compiler_params=_CP,
        scratch_types=[
            pltpu.VMEM((CH,), i32), pltpu.VMEM((CH,), i32),
            pltpu.VMEM((CH,), f32), pltpu.VMEM((CH,), f32), pltpu.VMEM((CH,), f32),
            pltpu.VMEM((CH, 8), f32), pltpu.VMEM((CH, 8), f32),
            pltpu.VMEM((CH, 8), f32),
            pltpu.VMEM_SHARED((NP, 8), f32),
        ],
    )
    def eb(ip, jp, qp, dxp, dyp, tbl, zin, out,
           ib, jb, qb, dxb, dyb, gi, gj, cb, acc):
        cid = lax.axis_index("c")
        sid = lax.axis_index("s")
        wid = sid * NC + cid
        NZ = NP // NS
        pltpu.sync_copy(zin, acc.at[pl.ds(sid * NZ, NZ)])
        zv = jnp.zeros((L,), f32)
        for g in range(G):
            ln = _lane(g)
            for c in range(2, 8):
                plsc.store_scatter(cb, [ln, _col(c)], zv)
        plsc.subcore_barrier()

        def chunk(t, carry):
            base = (wid * T + t) * CH
            pltpu.sync_copy(ip.at[pl.ds(base, CH)], ib)
            pltpu.sync_copy(jp.at[pl.ds(base, CH)], jb)
            pltpu.sync_copy(qp.at[pl.ds(base, CH)], qb)
            pltpu.sync_copy(dxp.at[pl.ds(base, CH)], dxb)
            pltpu.sync_copy(dyp.at[pl.ds(base, CH)], dyb)
            pltpu.sync_copy(tbl.at[ib], gi)
            pltpu.sync_copy(tbl.at[jb], gj)
            for g in range(G):
                s = g * L
                ln = _lane(g)
                q = qb[pl.ds(s, L)]
                gwx, gwy = _gradw(q, dxb[pl.ds(s, L)], dyb[pl.ds(s, L)])
                l00 = _lg(gi, ln, 0); l01 = _lg(gi, ln, 1)
                l10 = _lg(gi, ln, 2); l11 = _lg(gi, ln, 3)
                rhoi = _lg(gi, ln, 4)
                rhoj = _lg(gj, ln, 4); vj = _lg(gj, ln, 5)
                gx = l00 * gwx + l01 * gwy
                gy = l10 * gwx + l11 * gwy
                fac = ((rhoj - rhoi) * vj) * 2.0
                plsc.store_scatter(cb, [ln, _col(0)], fac * gx)
                plsc.store_scatter(cb, [ln, _col(1)], fac * gy)
            pltpu.sync_copy(cb, acc.at[ib], add=True)
            return carry

        lax.fori_loop(0, T, chunk, 0)
        plsc.subcore_barrier()
        pltpu.sync_copy(acc.at[pl.ds(sid * NZ, NZ)], out.at[cid, pl.ds(sid * NZ, NZ)])

    return eb


def _make_nb(NP, KN, mesh):
    @functools.partial(
        pl.kernel,
        out_type=jax.ShapeDtypeStruct((NP, 16), f32),
        mesh=mesh,
        compiler_params=_CP,
        scratch_types=[
            pltpu.VMEM((CH, 8), f32), pltpu.VMEM((CH, 8), f32),
            pltpu.VMEM((CH, 16), f32),
        ],
    )
    def nb(p01, t2pre, t2, pa, pb, tb):
        cid = lax.axis_index("c")
        sid = lax.axis_index("s")
        wid = sid * NC + cid

        def chunk(k, carry):
            r0 = (wid * KN + k) * CH
            pltpu.sync_copy(p01.at[0, pl.ds(r0, CH)], pa)
            pltpu.sync_copy(p01.at[1, pl.ds(r0, CH)], pb)
            pltpu.sync_copy(t2pre.at[pl.ds(r0, CH)], tb)
            for g in range(G):
                ln = _lane(g)
                plsc.store_scatter(tb, [ln, _col(7)],
                                   -(_lg(pa, ln, 0) + _lg(pb, ln, 0)))
                plsc.store_scatter(tb, [ln, _col(8)],
                                   -(_lg(pa, ln, 1) + _lg(pb, ln, 1)))
            pltpu.sync_copy(tb, t2.at[pl.ds(r0, CH)])
            return carry

        lax.fori_loop(0, KN, chunk, 0)

    return nb


def _make_e2(NP, T, mesh):
    @functools.partial(
        pl.kernel,
        out_type=jax.ShapeDtypeStruct((NC, NP, 8), f32),
        mesh=mesh,
        compiler_params=_CP,
        scratch_types=[
            pltpu.VMEM((CH,), i32), pltpu.VMEM((CH,), i32),
            pltpu.VMEM((CH,), f32), pltpu.VMEM((CH,), f32), pltpu.VMEM((CH,), f32),
            pltpu.VMEM((CH, 16), f32), pltpu.VMEM((CH, 16), f32),
            pltpu.VMEM((CH, 8), f32),
            pltpu.VMEM_SHARED((NP, 8), f32),
        ],
    )
    def e2(ip, jp, qp, dxp, dyp, t2, zin, out, ib, jb, qb, dxb, dyb, gi, gj, cb, acc):
        cid = lax.axis_index("c")
        sid = lax.axis_index("s")
        wid = sid * NC + cid
        NZ = NP // NS
        pltpu.sync_copy(zin, acc.at[pl.ds(sid * NZ, NZ)])
        zv = jnp.zeros((L,), f32)
        for g in range(G):
            ln = _lane(g)
            plsc.store_scatter(cb, [ln, _col(6)], zv)
            plsc.store_scatter(cb, [ln, _col(7)], zv)
        plsc.subcore_barrier()

        def chunk(t, carry):
            base = (wid * T + t) * CH
            pltpu.sync_copy(ip.at[pl.ds(base, CH)], ib)
            pltpu.sync_copy(jp.at[pl.ds(base, CH)], jb)
            pltpu.sync_copy(qp.at[pl.ds(base, CH)], qb)
            pltpu.sync_copy(dxp.at[pl.ds(base, CH)], dxb)
            pltpu.sync_copy(dyp.at[pl.ds(base, CH)], dyb)
            pltpu.sync_copy(t2.at[ib], gi)
            pltpu.sync_copy(t2.at[jb], gj)
            for g in range(G):
                s = g * L
                ln = _lane(g)
                q = qb[pl.ds(s, L)]
                gwx, gwy = _gradw(q, dxb[pl.ds(s, L)], dyb[pl.ds(s, L)])
                pxi = _lg(gi, ln, 0); pyi = _lg(gi, ln, 1)
                uxi = _lg(gi, ln, 2); uyi = _lg(gi, ln, 3)
                rhoi = _lg(gi, ln, 4); Pi = _lg(gi, ln, 6)
                rdgxi = _lg(gi, ln, 7); rdgyi = _lg(gi, ln, 8)
                pxj = _lg(gj, ln, 0); pyj = _lg(gj, ln, 1)
                uxj = _lg(gj, ln, 2); uyj = _lg(gj, ln, 3)
                rhoj = _lg(gj, ln, 4); vj = _lg(gj, ln, 5); Pj = _lg(gj, ln, 6)
                rdgxj = _lg(gj, ln, 7); rdgyj = _lg(gj, ln, 8)
                rjx = pxj - pxi
                rjy = pyj - pyi
                rji2 = (rjx * rjx + rjy * rjy) + EPS
                ujx = uxj - uxi
                ujy = uyj - uyi
                cp = (2.0 * (rhoj - rhoi)) / rji2
                psix = cp * rjx - (rdgxi + rdgxj)
                psiy = cp * rjy - (rdgyi + rdgyj)
                plsc.store_scatter(cb, [ln, _col(0)], (psix * gwx + psiy * gwy) * vj)
                plsc.store_scatter(cb, [ln, _col(1)],
                                   (ujx * gwx + ujy * gwy) * vj * rhoj)
                pi_ij = jnp.minimum(ujx * rjx + ujy * rjy, 0.0) / rji2
                vf = pi_ij * vj * rhoj / (rhoi + rhoj)
                plsc.store_scatter(cb, [ln, _col(2)], vf * gwx)
                plsc.store_scatter(cb, [ln, _col(3)], vf * gwy)
                pij = (Pi + Pj) * vj
                plsc.store_scatter(cb, [ln, _col(4)], pij * gwx)
                plsc.store_scatter(cb, [ln, _col(5)], pij * gwy)
            pltpu.sync_copy(cb, acc.at[ib], add=True)
            return carry

        lax.fori_loop(0, T, chunk, 0)
        plsc.subcore_barrier()
        pltpu.sync_copy(acc.at[pl.ds(sid * NZ, NZ)], out.at[cid, pl.ds(sid * NZ, NZ)])

    return e2


def _make_n2(NP, KN, mesh):
    @functools.partial(
        pl.kernel,
        out_type=(jax.ShapeDtypeStruct((NP,), f32),
                  jax.ShapeDtypeStruct((NP, 2), f32)),
        mesh=mesh,
        compiler_params=_CP,
        scratch_types=[
            pltpu.VMEM((CH, 8), f32), pltpu.VMEM((CH, 8), f32),
            pltpu.VMEM((CH,), f32),
            pltpu.VMEM((CH,), f32), pltpu.VMEM((CH, 2), f32),
        ],
    )
    def n2(p01, rfp, dp, du, pa, pb, rb_, db, ub):
        cid = lax.axis_index("c")
        sid = lax.axis_index("s")
        wid = sid * NC + cid

        def chunk(k, carry):
            r0 = (wid * KN + k) * CH
            pltpu.sync_copy(p01.at[0, pl.ds(r0, CH)], pa)
            pltpu.sync_copy(p01.at[1, pl.ds(r0, CH)], pb)
            pltpu.sync_copy(rfp.at[pl.ds(r0, CH)], rb_)
            for g in range(G):
                s = g * L
                ln = _lane(g)
                sdd = _lg(pa, ln, 0) + _lg(pb, ln, 0)
                sdv = _lg(pa, ln, 1) + _lg(pb, ln, 1)
                svx = _lg(pa, ln, 2) + _lg(pb, ln, 2)
                svy = _lg(pa, ln, 3) + _lg(pb, ln, 3)
                spx = _lg(pa, ln, 4) + _lg(pb, ln, 4)
                spy = _lg(pa, ln, 5) + _lg(pb, ln, 5)
                rfv = rb_[pl.ds(s, L)]
                db[pl.ds(s, L)] = (-sdv) + DDC * sdd
                mr = (-1.0) / rfv
                plsc.store_scatter(ub, [ln, _col(0)], mr * spx + VC * svx)
                plsc.store_scatter(ub, [ln, _col(1)], mr * spy + VC * svy)
            pltpu.sync_copy(db, dp.at[pl.ds(r0, CH)])
            pltpu.sync_copy(ub, du.at[pl.ds(r0, CH)])
            return carry

        lax.fori_loop(0, KN, chunk, 0)

    return n2


def kernel(i, j, fluidPosition, fluidVelocity, fluidDensity, fluidArea,
           fluidPressure, fluidDistances, fluidRadialDistances):
    N = fluidPosition.shape[0]
    E = i.shape[0]
    EPAD = -(-(E + 2 * CE) // (NW * CH)) * NW * CH
    T = EPAD // (NW * CH)
    KN = -(-N // (NW * CH))
    NP = KN * NW * CH
    NPG = NP + 16

    rhoF = fluidDensity * REST_DENSITY
    V = fluidArea * REST_DENSITY / fluidDensity / REST_DENSITY

    ii = i.astype(i32)
    pe = EPAD - E
    ip = jnp.concatenate([ii, jnp.full((pe,), N, i32)])
    jp = jnp.concatenate([j.astype(i32), jnp.zeros((pe,), i32)])
    qp = jnp.concatenate([fluidRadialDistances.astype(f32), jnp.ones((pe,), f32)])
    dxp = jnp.concatenate([fluidDistances[:, 0].astype(f32), jnp.zeros((pe,), f32)])
    dyp = jnp.concatenate([fluidDistances[:, 1].astype(f32), jnp.zeros((pe,), f32)])

    tgt = (jnp.arange(1, NW) * (E // NW)).astype(i32)
    cut = jnp.searchsorted(ii, ii[tgt], side="left").astype(i32)
    starts = jnp.concatenate([jnp.zeros((1,), i32), cut, jnp.full((1,), E, i32),
                              jnp.full((48 - NW - 1,), E, i32)])

    pn = NP - N

    def padn(x):
        return jnp.concatenate([x.astype(f32), jnp.zeros((pn,), f32)])

    px = padn(fluidPosition[:, 0]); py = padn(fluidPosition[:, 1])
    ux = padn(fluidVelocity[:, 0]); uy = padn(fluidVelocity[:, 1])
    rf = padn(rhoF); vv = padn(V); pp = padn(fluidPressure)
    zc = jnp.zeros((NP,), f32)
    zg = jnp.zeros((NPG - NP, 8), f32)
    t1 = jnp.concatenate([jnp.stack([px, py, rf, vv, zc, zc, zc, zc], axis=1), zg],
                         axis=0)
    tbpre = jnp.concatenate([jnp.stack([zc, zc, zc, zc, rf, vv, zc, zc], axis=1), zg],
                            axis=0)
    t2pre = jnp.stack([px, py, ux, uy, rf, vv, pp] + [zc] * 9, axis=1)

    mesh = plsc.VectorSubcoreMesh(core_axis_name="c", subcore_axis_name="s",
                                  num_cores=NC, num_subcores=NS)
    zin8g = jnp.zeros((NPG // NS, 8), f32)
    zin8 = jnp.zeros((NP // NS, 8), f32)

    p1 = _make_e1(NPG, mesh)(ip, jp, qp, dxp, dyp, t1, starts, zin8g)
    tbl = _make_n1(NPG, KN, mesh)(p1, tbpre)
    pb = _make_eb(NP, T, mesh)(ip, jp, qp, dxp, dyp, tbl, zin8)
    t2 = _make_nb(NP, KN, mesh)(pb, t2pre)
    p2 = _make_e2(NP, T, mesh)(ip, jp, qp, dxp, dyp, t2, zin8)
    dp, du = _make_n2(NP, KN, mesh)(p2, rf)
    return dp[:N], du[:N]
